# baseline (speedup 1.0000x reference)
import functools

import jax
import jax.numpy as jnp
from jax import lax
from jax.experimental import pallas as pl
from jax.experimental.pallas import tpu as pltpu
from jax.experimental.pallas import tpu_sc as plsc

T = 2048
E = 8
H = 2048
I = 4096
K = 2
B = 576
NT = 14
NSLOT = NT * B
BI = 512
NI = I // BI

NWORK = 32
PAIRS_PER_W = (K * T) // NWORK
CHUNK = 16
NCHUNK = PAIRS_PER_W // CHUNK
TOK_PER_W = T // NWORK


def _route_body(logits_ref, slot_ref, w_ref, te_ref):
    lg = logits_ref[...]
    ids = lax.broadcasted_iota(jnp.int32, (T, E), 1)
    m1 = jnp.max(lg, axis=1, keepdims=True)
    id1 = jnp.min(jnp.where(lg == m1, ids, E), axis=1)
    masked = jnp.where(ids == id1[:, None], -jnp.inf, lg)
    m2 = jnp.max(masked, axis=1, keepdims=True)
    id2 = jnp.min(jnp.where(masked == m2, ids, E), axis=1)
    d = m1[:, 0] - m2[:, 0]
    w1 = 1.0 / (1.0 + jnp.exp(-d))
    w2 = 1.0 / (1.0 + jnp.exp(d))
    w_ref[...] = jnp.stack([w1, w2], axis=0)

    e1 = (ids == id1[:, None]).astype(jnp.float32)
    e2 = (ids == id2[:, None]).astype(jnp.float32)
    ecat = jnp.concatenate([e1, e2], axis=0)
    csum = ecat
    sh = 1
    while sh < K * T:
        csum = csum + jnp.concatenate(
            [jnp.zeros((sh, E), jnp.float32), csum[: K * T - sh]], axis=0)
        sh *= 2
    counts = csum[K * T - 1 : K * T, :]
    ntiles = jnp.ceil(counts * (1.0 / B))
    tcum = ntiles
    for sh in (1, 2, 4):
        tcum = tcum + jnp.concatenate(
            [jnp.zeros((1, sh), jnp.float32), tcum[:, : E - sh]], axis=1)
    tstart = tcum - ntiles
    segstart = tstart * B
    slotf = jnp.sum(ecat * (segstart + csum - 1.0), axis=1)
    slot_ref[...] = slotf.reshape(K, T).astype(jnp.int32)

    tl = lax.broadcasted_iota(jnp.int32, (1, 128), 1).astype(jnp.float32)
    te = -jnp.ones((1, 128), jnp.float32)
    for e in range(E):
        te = te + (tl >= tstart[0, e]).astype(jnp.float32)
    te = jnp.clip(te, 0.0, E - 1)
    total = tcum[0, E - 1]
    e_last = jnp.sum(jnp.where(tl == total - 1.0, te, 0.0))
    phant = (tl >= total).astype(jnp.float32)
    te = jnp.where(phant > 0.0, e_last, te)
    tmap = jnp.minimum(tl, total - 1.0)
    te_ref[...] = jnp.concatenate([te, phant, tmap], axis=0).astype(jnp.int32)


_route = pl.pallas_call(
    _route_body,
    out_shape=(
        jax.ShapeDtypeStruct((K, T), jnp.int32),
        jax.ShapeDtypeStruct((K, T), jnp.float32),
        jax.ShapeDtypeStruct((3, 128), jnp.int32),
    ),
)


def _dispatch_body(x_hbm, slot_hbm, xs_hbm, idx2d, r0, r1,
                   sl0, sl1, ss0, ss1):
    wid = lax.axis_index("s") * 2 + lax.axis_index("c")
    base = wid * PAIRS_PER_W
    khalf = base // T
    t0 = base - khalf * T
    pltpu.sync_copy(slot_hbm.at[pl.ds(wid * NCHUNK, NCHUNK)], idx2d)

    rows = (r0, r1)
    lsem = (sl0, sl1)
    ssem = (ss0, ss1)
    pend = [None, None]
    for c in range(NCHUNK):
        b = c & 1
        if pend[b] is not None:
            pend[b].wait()
        ld = pltpu.async_copy(
            x_hbm.at[pl.ds(t0 + c * CHUNK, CHUNK)], rows[b], lsem[b])
        ld.wait()
        pend[b] = pltpu.async_copy(rows[b], xs_hbm.at[idx2d.at[c]], ssem[b])
    for b in (0, 1):
        if pend[b] is not None:
            pend[b].wait()


def _ffn_body(te_ref, xs_ref, wg_ref, wu_ref, w2_ref, ys_ref, h0_ref, h1_ref):
    t = pl.program_id(0)
    j = pl.program_id(1)
    real = te_ref[1, t] == 0
    dn = (((1,), (1,)), ((), ()))

    def gate_up(h_ref):
        x = xs_ref[...].astype(jnp.bfloat16)
        g = lax.dot_general(x, wg_ref[0].astype(jnp.bfloat16), dn,
                            preferred_element_type=jnp.float32)
        u = lax.dot_general(x, wu_ref[0].astype(jnp.bfloat16), dn,
                            preferred_element_type=jnp.float32)
        h_ref[...] = (g * lax.logistic(g) * u).astype(jnp.bfloat16)

    def down(h_ref):
        o = lax.dot_general(h_ref[...], w2_ref[0].astype(jnp.bfloat16), dn,
                            preferred_element_type=jnp.float32)

        @pl.when(j == 1)
        def _():
            ys_ref[...] = o

        @pl.when(j > 1)
        def _():
            ys_ref[...] += o

    @pl.when(real & (j < NI) & (j % 2 == 0))
    def _():
        gate_up(h0_ref)

    @pl.when(real & (j < NI) & (j % 2 == 1))
    def _():
        gate_up(h1_ref)

    @pl.when(real & (j >= 1) & (j % 2 == 1))
    def _():
        down(h0_ref)

    @pl.when(real & (j >= 1) & (j % 2 == 0))
    def _():
        down(h1_ref)


def _wi(j, te, t):
    return jnp.where(te[1, t] == 1, NI - 1, jnp.minimum(j, NI - 1))


def _wv(j, te, t):
    return jnp.where(te[1, t] == 1, NI - 1, jnp.maximum(j - 1, 0))


_ffn = pl.pallas_call(
    _ffn_body,
    grid_spec=pltpu.PrefetchScalarGridSpec(
        num_scalar_prefetch=1,
        grid=(NT, NI + 1),
        in_specs=[
            pl.BlockSpec((B, H), lambda t, j, te: (te[2, t], 0)),
            pl.BlockSpec((1, BI, H), lambda t, j, te: (te[0, t], _wi(j, te, t), 0)),
            pl.BlockSpec((1, BI, H), lambda t, j, te: (te[0, t], _wi(j, te, t) + NI, 0)),
            pl.BlockSpec((1, H, BI), lambda t, j, te: (te[0, t], 0, _wv(j, te, t))),
        ],
        out_specs=pl.BlockSpec((B, H), lambda t, j, te: (te[2, t], 0)),
        scratch_shapes=[
            pltpu.VMEM((B, BI), jnp.bfloat16),
            pltpu.VMEM((B, BI), jnp.bfloat16),
        ],
    ),
    out_shape=jax.ShapeDtypeStruct((NSLOT, H), jnp.float32),
    compiler_params=pltpu.CompilerParams(
        dimension_semantics=("arbitrary", "arbitrary"),
        vmem_limit_bytes=128 * 1024 * 1024,
    ),
)


CC = 8
NCC = TOK_PER_W // CC


def _combine_body(ys_hbm, slot_hbm, w_hbm, out_hbm,
                  idx1, idx2, wv, b1a, b1b, b2a, b2b,
                  g1a, g1b, g2a, g2b, so0, so1):
    wid = lax.axis_index("s") * 2 + lax.axis_index("c")
    t0 = wid * TOK_PER_W
    pltpu.sync_copy(slot_hbm.at[0, pl.ds(wid * NCC, NCC)], idx1)
    pltpu.sync_copy(slot_hbm.at[1, pl.ds(wid * NCC, NCC)], idx2)
    pltpu.sync_copy(w_hbm.at[:, pl.ds(wid * (TOK_PER_W // 16), 4)], wv)

    b1 = (b1a, b1b)
    b2 = (b2a, b2b)
    g1s = (g1a, g1b)
    g2s = (g2a, g2b)
    osem = (so0, so1)

    def gather(c, b):
        return (pltpu.async_copy(ys_hbm.at[idx1.at[c]], b1[b], g1s[b]),
                pltpu.async_copy(ys_hbm.at[idx2.at[c]], b2[b], g2s[b]))

    pend_out = [None, None]
    g = [gather(0, 0), None]
    for c in range(NCC):
        b = c & 1
        nb = (c + 1) & 1
        if c + 1 < NCC:
            if pend_out[nb] is not None:
                pend_out[nb].wait()
                pend_out[nb] = None
            g[nb] = gather(c + 1, nb)
        g[b][0].wait()
        g[b][1].wait()
        w1x = wv[0, (c * CC) // 16, pl.ds(0, 16)]
        w2x = wv[1, (c * CC) // 16, pl.ds(0, 16)]
        roff = (c * CC) % 16

        def col(v, inner):
            for r in range(CC):
                a = b1[b][r, pl.ds(v * 16, 16)]
                d = b2[b][r, pl.ds(v * 16, 16)]
                b1[b][r, pl.ds(v * 16, 16)] = (
                    w1x[roff + r] * a + w2x[roff + r] * d)
            return inner

        lax.fori_loop(0, H // 16, col, 0)
        pend_out[b] = pltpu.async_copy(
            b1[b], out_hbm.at[pl.ds(t0 + c * CC, CC)], osem[b])
    for b in (0, 1):
        if pend_out[b] is not None:
            pend_out[b].wait()


@functools.lru_cache(maxsize=1)
def _sc_kernels():
    mesh = plsc.VectorSubcoreMesh(core_axis_name="c", subcore_axis_name="s")
    dispatch = pl.kernel(
        _dispatch_body,
        mesh=mesh,
        out_type=jax.ShapeDtypeStruct((NSLOT, H), jnp.float32),
        scratch_types=[
            pltpu.VMEM((NCHUNK, CHUNK), jnp.int32),
            pltpu.VMEM((CHUNK, H), jnp.float32),
            pltpu.VMEM((CHUNK, H), jnp.float32),
            pltpu.SemaphoreType.DMA,
            pltpu.SemaphoreType.DMA,
            pltpu.SemaphoreType.DMA,
            pltpu.SemaphoreType.DMA,
        ],
    )
    combine = pl.kernel(
        _combine_body,
        mesh=mesh,
        out_type=jax.ShapeDtypeStruct((T, H), jnp.float32),
        scratch_types=[
            pltpu.VMEM((NCC, CC), jnp.int32),
            pltpu.VMEM((NCC, CC), jnp.int32),
            pltpu.VMEM((2, TOK_PER_W // 16, 16), jnp.float32),
            pltpu.VMEM((CC, H), jnp.float32),
            pltpu.VMEM((CC, H), jnp.float32),
            pltpu.VMEM((CC, H), jnp.float32),
            pltpu.VMEM((CC, H), jnp.float32),
            pltpu.SemaphoreType.DMA,
            pltpu.SemaphoreType.DMA,
            pltpu.SemaphoreType.DMA,
            pltpu.SemaphoreType.DMA,
            pltpu.SemaphoreType.DMA,
            pltpu.SemaphoreType.DMA,
        ],
    )
    return dispatch, combine


def kernel(hidden_states, router_logits, w13_weight, w2_weight):
    dispatch, combine = _sc_kernels()
    slot, w, te = _route(router_logits)
    xs = dispatch(hidden_states, slot.reshape(K * T // CHUNK, CHUNK))
    ys = _ffn(te, xs, w13_weight, w13_weight, w2_weight)
    return combine(ys, slot.reshape(K, T // CC, CC),
                   w.reshape(K, T // 16, 16))


# speedup vs baseline: 1.1172x; 1.1172x over previous
import functools

import jax
import jax.numpy as jnp
from jax import lax
from jax.experimental import pallas as pl
from jax.experimental.pallas import tpu as pltpu
from jax.experimental.pallas import tpu_sc as plsc

T = 2048
E = 8
H = 2048
I = 4096
K = 2
B = 576
NT = 14
NSLOT = NT * B
BI = 512
NI = I // BI

NWORK = 32
PAIRS_PER_W = (K * T) // NWORK
CHUNK = 16
NCHUNK = PAIRS_PER_W // CHUNK
TOK_PER_W = T // NWORK


def _route_body(logits_ref, slot_ref, w_ref, te_ref):
    lg = logits_ref[...]
    ids = lax.broadcasted_iota(jnp.int32, (T, E), 1)
    m1 = jnp.max(lg, axis=1, keepdims=True)
    id1 = jnp.min(jnp.where(lg == m1, ids, E), axis=1)
    masked = jnp.where(ids == id1[:, None], -jnp.inf, lg)
    m2 = jnp.max(masked, axis=1, keepdims=True)
    id2 = jnp.min(jnp.where(masked == m2, ids, E), axis=1)
    d = m1[:, 0] - m2[:, 0]
    w1 = 1.0 / (1.0 + jnp.exp(-d))
    w2 = 1.0 / (1.0 + jnp.exp(d))
    w_ref[...] = jnp.stack([w1, w2], axis=0)

    e1 = (ids == id1[:, None]).astype(jnp.float32)
    e2 = (ids == id2[:, None]).astype(jnp.float32)
    ecat = jnp.concatenate([e1, e2], axis=0)
    csum = ecat
    sh = 1
    while sh < K * T:
        csum = csum + jnp.concatenate(
            [jnp.zeros((sh, E), jnp.float32), csum[: K * T - sh]], axis=0)
        sh *= 2
    counts = csum[K * T - 1 : K * T, :]
    ntiles = jnp.ceil(counts * (1.0 / B))
    tcum = ntiles
    for sh in (1, 2, 4):
        tcum = tcum + jnp.concatenate(
            [jnp.zeros((1, sh), jnp.float32), tcum[:, : E - sh]], axis=1)
    tstart = tcum - ntiles
    segstart = tstart * B
    slotf = jnp.sum(ecat * (segstart + csum - 1.0), axis=1)
    slot_ref[...] = slotf.reshape(K, T).astype(jnp.int32)

    tl = lax.broadcasted_iota(jnp.int32, (1, 128), 1).astype(jnp.float32)
    te = -jnp.ones((1, 128), jnp.float32)
    for e in range(E):
        te = te + (tl >= tstart[0, e]).astype(jnp.float32)
    te = jnp.clip(te, 0.0, E - 1)
    total = tcum[0, E - 1]
    e_last = jnp.sum(jnp.where(tl == total - 1.0, te, 0.0))
    phant = (tl >= total).astype(jnp.float32)
    te = jnp.where(phant > 0.0, e_last, te)
    tmap = jnp.minimum(tl, total - 1.0)
    te_ref[...] = jnp.concatenate([te, phant, tmap], axis=0).astype(jnp.int32)


_route = pl.pallas_call(
    _route_body,
    out_shape=(
        jax.ShapeDtypeStruct((K, T), jnp.int32),
        jax.ShapeDtypeStruct((K, T), jnp.float32),
        jax.ShapeDtypeStruct((3, 128), jnp.int32),
    ),
)


def _dispatch_body(x_hbm, slot_hbm, xs_hbm, idx2d, r0, r1,
                   sl0, sl1, ss0, ss1):
    wid = lax.axis_index("s") * 2 + lax.axis_index("c")
    base = wid * PAIRS_PER_W
    khalf = base // T
    t0 = base - khalf * T
    pltpu.sync_copy(slot_hbm.at[pl.ds(wid * NCHUNK, NCHUNK)], idx2d)

    rows = (r0, r1)
    lsem = (sl0, sl1)
    ssem = (ss0, ss1)
    pend = [None, None]
    for c in range(NCHUNK):
        b = c & 1
        if pend[b] is not None:
            pend[b].wait()
        ld = pltpu.async_copy(
            x_hbm.at[pl.ds(t0 + c * CHUNK, CHUNK)], rows[b], lsem[b])
        ld.wait()
        pend[b] = pltpu.async_copy(rows[b], xs_hbm.at[idx2d.at[c]], ssem[b])
    for b in (0, 1):
        if pend[b] is not None:
            pend[b].wait()


def _ffn_body(te_ref, xs_ref, wg_ref, wu_ref, w2_ref, ys_ref):
    t = pl.program_id(0)
    i = pl.program_id(1)

    @pl.when(te_ref[1, t] == 0)
    def _():
        x = xs_ref[...].astype(jnp.bfloat16)
        dn = (((1,), (1,)), ((), ()))
        g = lax.dot_general(x, wg_ref[0].astype(jnp.bfloat16), dn,
                            preferred_element_type=jnp.float32)
        u = lax.dot_general(x, wu_ref[0].astype(jnp.bfloat16), dn,
                            preferred_element_type=jnp.float32)
        h = (g * lax.logistic(g) * u).astype(jnp.bfloat16)
        o = lax.dot_general(h, w2_ref[0].astype(jnp.bfloat16), dn,
                            preferred_element_type=jnp.float32)

        @pl.when(i == 0)
        def _():
            ys_ref[...] = o

        @pl.when(i > 0)
        def _():
            ys_ref[...] += o


def _wi(i, te, t):
    return jnp.where(te[1, t] == 1, NI - 1, i)


_ffn = pl.pallas_call(
    _ffn_body,
    grid_spec=pltpu.PrefetchScalarGridSpec(
        num_scalar_prefetch=1,
        grid=(NT, NI),
        in_specs=[
            pl.BlockSpec((B, H), lambda t, i, te: (te[2, t], 0)),
            pl.BlockSpec((1, BI, H), lambda t, i, te: (te[0, t], _wi(i, te, t), 0)),
            pl.BlockSpec((1, BI, H), lambda t, i, te: (te[0, t], _wi(i, te, t) + NI, 0)),
            pl.BlockSpec((1, H, BI), lambda t, i, te: (te[0, t], 0, _wi(i, te, t))),
        ],
        out_specs=pl.BlockSpec((B, H), lambda t, i, te: (te[2, t], 0)),
    ),
    out_shape=jax.ShapeDtypeStruct((NSLOT, H), jnp.float32),
    compiler_params=pltpu.CompilerParams(
        dimension_semantics=("arbitrary", "arbitrary"),
        vmem_limit_bytes=128 * 1024 * 1024,
    ),
)


CC = 8
NCC = TOK_PER_W // CC


def _combine_body(ys_hbm, slot_hbm, w_hbm, out_hbm,
                  idx1, idx2, wv, b1a, b1b, b2a, b2b,
                  g1a, g1b, g2a, g2b, so0, so1):
    wid = lax.axis_index("s") * 2 + lax.axis_index("c")
    t0 = wid * TOK_PER_W
    pltpu.sync_copy(slot_hbm.at[0, pl.ds(wid * NCC, NCC)], idx1)
    pltpu.sync_copy(slot_hbm.at[1, pl.ds(wid * NCC, NCC)], idx2)
    pltpu.sync_copy(w_hbm.at[:, pl.ds(wid * (TOK_PER_W // 16), 4)], wv)

    b1 = (b1a, b1b)
    b2 = (b2a, b2b)
    g1s = (g1a, g1b)
    g2s = (g2a, g2b)
    osem = (so0, so1)

    def gather(c, b):
        return (pltpu.async_copy(ys_hbm.at[idx1.at[c]], b1[b], g1s[b]),
                pltpu.async_copy(ys_hbm.at[idx2.at[c]], b2[b], g2s[b]))

    pend_out = [None, None]
    g = [gather(0, 0), None]
    for c in range(NCC):
        b = c & 1
        nb = (c + 1) & 1
        if c + 1 < NCC:
            if pend_out[nb] is not None:
                pend_out[nb].wait()
                pend_out[nb] = None
            g[nb] = gather(c + 1, nb)
        g[b][0].wait()
        g[b][1].wait()
        w1x = wv[0, (c * CC) // 16, pl.ds(0, 16)]
        w2x = wv[1, (c * CC) // 16, pl.ds(0, 16)]
        roff = (c * CC) % 16

        def col(v, inner):
            for r in range(CC):
                a = b1[b][r, pl.ds(v * 16, 16)]
                d = b2[b][r, pl.ds(v * 16, 16)]
                b1[b][r, pl.ds(v * 16, 16)] = (
                    w1x[roff + r] * a + w2x[roff + r] * d)
            return inner

        lax.fori_loop(0, H // 16, col, 0)
        pend_out[b] = pltpu.async_copy(
            b1[b], out_hbm.at[pl.ds(t0 + c * CC, CC)], osem[b])
    for b in (0, 1):
        if pend_out[b] is not None:
            pend_out[b].wait()


@functools.lru_cache(maxsize=1)
def _sc_kernels():
    mesh = plsc.VectorSubcoreMesh(core_axis_name="c", subcore_axis_name="s")
    dispatch = pl.kernel(
        _dispatch_body,
        mesh=mesh,
        out_type=jax.ShapeDtypeStruct((NSLOT, H), jnp.float32),
        scratch_types=[
            pltpu.VMEM((NCHUNK, CHUNK), jnp.int32),
            pltpu.VMEM((CHUNK, H), jnp.float32),
            pltpu.VMEM((CHUNK, H), jnp.float32),
            pltpu.SemaphoreType.DMA,
            pltpu.SemaphoreType.DMA,
            pltpu.SemaphoreType.DMA,
            pltpu.SemaphoreType.DMA,
        ],
    )
    combine = pl.kernel(
        _combine_body,
        mesh=mesh,
        out_type=jax.ShapeDtypeStruct((T, H), jnp.float32),
        scratch_types=[
            pltpu.VMEM((NCC, CC), jnp.int32),
            pltpu.VMEM((NCC, CC), jnp.int32),
            pltpu.VMEM((2, TOK_PER_W // 16, 16), jnp.float32),
            pltpu.VMEM((CC, H), jnp.float32),
            pltpu.VMEM((CC, H), jnp.float32),
            pltpu.VMEM((CC, H), jnp.float32),
            pltpu.VMEM((CC, H), jnp.float32),
            pltpu.SemaphoreType.DMA,
            pltpu.SemaphoreType.DMA,
            pltpu.SemaphoreType.DMA,
            pltpu.SemaphoreType.DMA,
            pltpu.SemaphoreType.DMA,
            pltpu.SemaphoreType.DMA,
        ],
    )
    return dispatch, combine


def kernel(hidden_states, router_logits, w13_weight, w2_weight):
    dispatch, combine = _sc_kernels()
    slot, w, te = _route(router_logits)
    xs = dispatch(hidden_states, slot.reshape(K * T // CHUNK, CHUNK))
    ys = _ffn(te, xs, w13_weight, w13_weight, w2_weight)
    return combine(ys, slot.reshape(K, T // CC, CC),
                   w.reshape(K, T // 16, 16))
